# baseline (speedup 1.0000x reference)
import jax
import jax.numpy as jnp
from jax import lax
from jax.experimental import pallas as pl
from jax.experimental.pallas import tpu as pltpu
from jax.experimental.pallas import tpu_sc as plsc

_BLK = 8000
_BIG = 2**30


def _tc_body(mesh_ref, recv_ref, out_ref, idx_ref, minval, minidx):
    i = pl.program_id(0)

    @pl.when(i == 0)
    def _init():
        minval[...] = jnp.full_like(minval, jnp.inf)
        minidx[...] = jnp.zeros_like(minidx)

    xb = mesh_ref[:, 0:1]
    yb = mesh_ref[:, 1:2]
    zb = mesh_ref[:, 2:3]
    rx = recv_ref[0:1, 0:32]
    ry = recv_ref[1:2, 0:32]
    rz = recv_ref[2:3, 0:32]
    d2 = (xb - rx) ** 2 + (yb - ry) ** 2 + (zb - rz) ** 2

    bmin = jnp.min(d2, axis=0, keepdims=True)
    rows = lax.broadcasted_iota(jnp.int32, d2.shape, 0)
    bidx = jnp.min(jnp.where(d2 <= bmin, rows, _BIG), axis=0, keepdims=True)
    gidx = bidx + i * _BLK

    better = bmin < minval[...]
    minval[...] = jnp.where(better, bmin, minval[...])
    minidx[...] = jnp.where(better, gidx, minidx[...])

    out_ref[:, 0:3] = mesh_ref[...]
    out_ref[:, 3:4] = jnp.zeros((_BLK, 1), jnp.float32)
    idx_ref[0:1, 0:32] = minidx[...]


def _tc_scan(mesh_flat, recv_pad, n):
    grid = n // _BLK
    out, idx = pl.pallas_call(
        _tc_body,
        grid=(grid,),
        in_specs=[
            pl.BlockSpec((_BLK, 3), lambda i: (i, 0)),
            pl.BlockSpec((8, 128), lambda i: (0, 0)),
        ],
        out_specs=[
            pl.BlockSpec((_BLK, 4), lambda i: (i, 0)),
            pl.BlockSpec((8, 128), lambda i: (0, 0)),
        ],
        out_shape=[
            jax.ShapeDtypeStruct((n, 4), jnp.float32),
            jax.ShapeDtypeStruct((8, 128), jnp.int32),
        ],
        scratch_shapes=[
            pltpu.VMEM((1, 32), jnp.float32),
            pltpu.VMEM((1, 32), jnp.int32),
        ],
        compiler_params=pltpu.CompilerParams(
            dimension_semantics=("arbitrary",),
        ),
    )(mesh_flat, recv_pad)
    return out, idx


def _sc_finale_body(minidx_hbm, out_rows_ref, closest_hbm,
                    idx_v, rid_v, pos_v, base_v, buf_v, cl_v, sem):
    cid = lax.axis_index("c")
    sid = lax.axis_index("s")

    @pl.when((cid == 0) & (sid == 0))
    def _():
        pltpu.sync_copy(minidx_hbm, idx_v)
        for k in range(2):
            sl = pl.ds(16 * k, 16)
            w = idx_v[sl] * 4 + 3
            rid_v[sl] = lax.shift_right_logical(w, 7)
            pos_v[sl] = lax.bitwise_and(w, 127)
            base_v[sl] = lax.bitwise_and(idx_v[sl] * 4, 127)
        pltpu.async_copy(out_rows_ref.at[rid_v], buf_v, sem).wait()
        for c in range(3):
            for k in range(2):
                sl = pl.ds(16 * k, 16)
                rows16 = lax.iota(jnp.int32, 16) + 16 * k
                vals = plsc.load_gather(buf_v, [rows16, base_v[sl] + c])
                plsc.store_scatter(cl_v, [rows16, jnp.full((16,), c, jnp.int32)],
                                   vals)
        pltpu.sync_copy(cl_v, closest_hbm)
        ones16 = jnp.full((16,), 1.0, jnp.float32)
        for j in range(32):
            j16 = jnp.full((16,), j, jnp.int32)
            rj = plsc.load_gather(rid_v, [j16])
            for k in range(2):
                sl = pl.ds(16 * k, 16)
                m = rid_v[sl] == rj
                plsc.store_scatter(buf_v, [j16, pos_v[sl]], ones16, mask=m)
        pltpu.async_copy(buf_v, out_rows_ref.at[rid_v], sem).wait()


import functools


@functools.cache
def _sc_finale():
    return pl.kernel(
        _sc_finale_body,
        out_type=jax.ShapeDtypeStruct((32, 3), jnp.float32),
        mesh=plsc.VectorSubcoreMesh(core_axis_name="c", subcore_axis_name="s"),
        scratch_types=[
            pltpu.VMEM((32,), jnp.int32),
            pltpu.VMEM((32,), jnp.int32),
            pltpu.VMEM((32,), jnp.int32),
            pltpu.VMEM((32,), jnp.int32),
            pltpu.VMEM((32, 128), jnp.float32),
            pltpu.VMEM((32, 3), jnp.float32),
            pltpu.SemaphoreType.DMA,
        ],
        compiler_params=pltpu.CompilerParams(needs_layout_passes=False),
    )


@jax.jit
def kernel(mesh_3D, receiver_pos):
    mesh_flat = mesh_3D.reshape(-1, 3)
    n = mesh_flat.shape[0]
    recv_pad = jnp.zeros((8, 128), jnp.float32).at[:3, :32].set(receiver_pos.T)

    out, idx_pad = _tc_scan(mesh_flat, recv_pad, n)
    minidx = idx_pad[0, :32]

    out_ref = jax.new_ref(out.reshape(n * 4 // 128, 128))
    closest = _sc_finale()(minidx, out_ref)
    return out_ref[...].reshape(n, 4), closest
